# Initial kernel scaffold; baseline (speedup 1.0000x reference)
import jax
import jax.numpy as jnp
from jax.experimental import pallas as pl


def kernel(x, edge_index, edge_attr, intarna_energy, batch, covalent_edges, dropout_conv_1_2, dropout_conv_rest, c1_eW, c1_eb, c1_preW, c1_preb, c1_postW, c1_postb, c1_linW, c1_linb, c1_gamma, c1_beta, c2_eW, c2_eb, c2_preW, c2_preb, c2_postW, c2_postb, c2_linW, c2_linb, c2_gamma, c2_beta, c3_eW, c3_eb, c3_preW, c3_preb, c3_postW, c3_postb, c3_linW, c3_linb, c3_gamma, c3_beta, c4_eW, c4_eb, c4_preW, c4_preb, c4_postW, c4_postb, c4_linW, c4_linb, c4_gamma, c4_beta, lin1_W, lin1_b, lin2_W, lin2_b, lin3_W, lin3_b):
    raise NotImplementedError("write your pallas kernel here")


# speedup vs baseline: 1.2262x; 1.2262x over previous
import functools
import math

import jax
import jax.numpy as jnp
from jax.experimental import pallas as pl
from jax.experimental.pallas import tpu as pltpu

N = 10000
E = 160000
D = 128
G = 64
AVG_LOG = math.log(17.0)
BLK = 1000


def _pre_body(use_bn, x_ref, cs_ref, cq_ref, g_ref, bb_ref, p1_ref, p2_ref,
              cvec_ref, xhat_ref, a_ref, bt_ref):
    xv = x_ref[...]
    if use_bn:
        m = cs_ref[...] * (1.0 / N)
        v = cq_ref[...] * (1.0 / N) - m * m
        xv = (xv - m) / jnp.sqrt(v + 1e-5) * g_ref[...] + bb_ref[...]
        xv = jnp.maximum(xv, 0.0)
    xhat_ref[...] = xv
    a_ref[...] = jnp.dot(xv, p1_ref[...], preferred_element_type=jnp.float32)
    bt_ref[...] = (jnp.dot(xv, p2_ref[...], preferred_element_type=jnp.float32)
                   + cvec_ref[...])


def _tc_pre(x, cs, cq, gamma, beta, p1, p2, cvec, use_bn):
    row = lambda i: (i, 0)
    full = lambda i: (0, 0)
    return pl.pallas_call(
        functools.partial(_pre_body, use_bn),
        grid=(N // BLK,),
        in_specs=[
            pl.BlockSpec((BLK, D), row),
            pl.BlockSpec((1, D), full),
            pl.BlockSpec((1, D), full),
            pl.BlockSpec((1, D), full),
            pl.BlockSpec((1, D), full),
            pl.BlockSpec((D, D), full),
            pl.BlockSpec((D, D), full),
            pl.BlockSpec((1, D), full),
        ],
        out_specs=[
            pl.BlockSpec((BLK, D), row),
            pl.BlockSpec((BLK, D), row),
            pl.BlockSpec((BLK, D), row),
        ],
        out_shape=[jax.ShapeDtypeStruct((N, D), jnp.float32)] * 3,
    )(x, cs, cq, gamma, beta, p1, p2, cvec)


def _post_body(x_ref, a_ref, s1_ref, s2_ref, tn_ref, tx_ref, cnt_ref,
               q0_ref, q1_ref, q2_ref, q3_ref, pb_ref, lw_ref, lb_ref,
               y_ref, cs_ref, cq_ref):
    i = pl.program_id(0)
    a = a_ref[...]
    s1 = s1_ref[...]
    s2 = s2_ref[...]
    cnt = cnt_ref[...]
    cntc = jnp.maximum(cnt, 1.0)
    inv = 1.0 / cntc
    mean = (cnt * a + s1) * inv
    s2m = (cnt * a * a + 2.0 * a * s1 + s2) * inv
    var = s2m - mean * mean
    std = jnp.sqrt(jnp.maximum(var, 1e-5))
    has = cnt > 0.0
    mn = jnp.where(has, a + tn_ref[...], 0.0)
    mx = jnp.where(has, a + tx_ref[...], 0.0)
    A = jnp.concatenate([mean, mn, mx, std], axis=-1)
    lg = jnp.log(cntc + 1.0)
    amp = lg * (1.0 / AVG_LOG)
    att = AVG_LOG / lg
    dot = lambda u, w: jnp.dot(u, w[...], preferred_element_type=jnp.float32)
    z = (dot(x_ref[...], q0_ref) + dot(A, q1_ref) + dot(A * amp, q2_ref)
         + dot(A * att, q3_ref) + pb_ref[...])
    y = dot(z, lw_ref) + lb_ref[...]
    y_ref[...] = y

    @pl.when(i == 0)
    def _():
        cs_ref[...] = jnp.zeros_like(cs_ref)
        cq_ref[...] = jnp.zeros_like(cq_ref)

    cs_ref[...] += jnp.sum(y, axis=0, keepdims=True)
    cq_ref[...] += jnp.sum(y * y, axis=0, keepdims=True)


def _tc_post(x, a, s1, s2, tn, tx, cnt, q0, q1, q2, q3, pb, lw, lb):
    row = lambda i: (i, 0)
    full = lambda i: (0, 0)
    return pl.pallas_call(
        _post_body,
        grid=(N // BLK,),
        in_specs=[
            pl.BlockSpec((BLK, D), row),
            pl.BlockSpec((BLK, D), row),
            pl.BlockSpec((BLK, D), row),
            pl.BlockSpec((BLK, D), row),
            pl.BlockSpec((BLK, D), row),
            pl.BlockSpec((BLK, D), row),
            pl.BlockSpec((BLK, 1), row),
            pl.BlockSpec((D, D), full),
            pl.BlockSpec((4 * D, D), full),
            pl.BlockSpec((4 * D, D), full),
            pl.BlockSpec((4 * D, D), full),
            pl.BlockSpec((1, D), full),
            pl.BlockSpec((D, D), full),
            pl.BlockSpec((1, D), full),
        ],
        out_specs=[
            pl.BlockSpec((BLK, D), row),
            pl.BlockSpec((1, D), full),
            pl.BlockSpec((1, D), full),
        ],
        out_shape=[
            jax.ShapeDtypeStruct((N, D), jnp.float32),
            jax.ShapeDtypeStruct((1, D), jnp.float32),
            jax.ShapeDtypeStruct((1, D), jnp.float32),
        ],
    )(x, a, s1, s2, tn, tx, cnt, q0, q1, q2, q3, pb, lw, lb)


def _norm_body(x_ref, cs_ref, cq_ref, g_ref, bb_ref, o_ref):
    m = cs_ref[...] * (1.0 / N)
    v = cq_ref[...] * (1.0 / N) - m * m
    o_ref[...] = jnp.maximum(
        (x_ref[...] - m) / jnp.sqrt(v + 1e-5) * g_ref[...] + bb_ref[...], 0.0)


def _tc_norm(x, cs, cq, gamma, beta):
    row = lambda i: (i, 0)
    full = lambda i: (0, 0)
    return pl.pallas_call(
        _norm_body,
        grid=(N // BLK,),
        in_specs=[
            pl.BlockSpec((BLK, D), row),
            pl.BlockSpec((1, D), full),
            pl.BlockSpec((1, D), full),
            pl.BlockSpec((1, D), full),
            pl.BlockSpec((1, D), full),
        ],
        out_specs=pl.BlockSpec((BLK, D), row),
        out_shape=jax.ShapeDtypeStruct((N, D), jnp.float32),
    )(x, cs, cq, gamma, beta)


def _mlp_body(xa_ref, xs_ref, cg_ref, w1_ref, b1_ref, w2_ref, b2_ref,
              w3_ref, b3_ref, o_ref):
    cg = cg_ref[...]
    xa = jnp.where(cg > 0.0, xa_ref[...], 0.0)
    xb = xs_ref[...] / jnp.maximum(cg, 1.0)
    h = jnp.concatenate([xa, xb], axis=1)
    dot = lambda u, w: jnp.dot(u, w[...], preferred_element_type=jnp.float32)
    h = jnp.maximum(dot(h, w1_ref) + b1_ref[...], 0.0)
    h = jnp.maximum(dot(h, w2_ref) + b2_ref[...], 0.0)
    o_ref[...] = dot(h, w3_ref) + b3_ref[...]


def _tc_mlp(xa, xs, cg, w1, b1, w2, b2, w3p, b3p):
    return pl.pallas_call(
        _mlp_body,
        out_shape=jax.ShapeDtypeStruct((G, D), jnp.float32),
    )(xa, xs, cg, w1, b1, w2, b2, w3p, b3p)


def kernel(x, edge_index, edge_attr, intarna_energy, batch, covalent_edges,
           dropout_conv_1_2, dropout_conv_rest,
           c1_eW, c1_eb, c1_preW, c1_preb, c1_postW, c1_postb, c1_linW,
           c1_linb, c1_gamma, c1_beta,
           c2_eW, c2_eb, c2_preW, c2_preb, c2_postW, c2_postb, c2_linW,
           c2_linb, c2_gamma, c2_beta,
           c3_eW, c3_eb, c3_preW, c3_preb, c3_postW, c3_postb, c3_linW,
           c3_linb, c3_gamma, c3_beta,
           c4_eW, c4_eb, c4_preW, c4_preb, c4_postW, c4_postb, c4_linW,
           c4_linb, c4_gamma, c4_beta,
           lin1_W, lin1_b, lin2_W, lin2_b, lin3_W, lin3_b):
    src = edge_index[0]
    dst = edge_index[1]

    key = (dst.astype(jnp.uint32) << 18) | jnp.arange(E, dtype=jnp.uint32)
    skey = jax.lax.sort(key)
    perm = (skey & jnp.uint32((1 << 18) - 1)).astype(jnp.int32)
    sdst = (skey >> 18).astype(jnp.int32)
    row_start = jnp.searchsorted(sdst, jnp.arange(N + 1, dtype=jnp.int32))
    cnt = (row_start[1:] - row_start[:-1]).astype(jnp.float32)[:, None]

    layers = [
        (c1_eW, c1_eb, c1_preW, c1_preb, c1_postW, c1_postb, c1_linW,
         c1_linb, c1_gamma, c1_beta),
        (c2_eW, c2_eb, c2_preW, c2_preb, c2_postW, c2_postb, c2_linW,
         c2_linb, c2_gamma, c2_beta),
        (c3_eW, c3_eb, c3_preW, c3_preb, c3_postW, c3_postb, c3_linW,
         c3_linb, c3_gamma, c3_beta),
        (c4_eW, c4_eb, c4_preW, c4_preb, c4_postW, c4_postb, c4_linW,
         c4_linb, c4_gamma, c4_beta),
    ]

    r2 = lambda v: v.reshape(1, D)
    xcur = x
    cs = jnp.zeros((1, D), jnp.float32)
    cq = jnp.zeros((1, D), jnp.float32)
    for i, (eW, eb, preW, preb, postW, postb, linW, linb, gamma,
            beta) in enumerate(layers):
        p1 = preW[:D]
        p2 = preW[D:2 * D]
        p3 = preW[2 * D:]
        w4 = eW @ p3
        cvec = (eb @ p3 + preb).reshape(1, D)
        q0 = postW[:D]
        q1 = postW[D:5 * D]
        q2 = postW[5 * D:9 * D]
        q3 = postW[9 * D:]

        xhat, a, btab = _tc_pre(xcur, cs, cq, r2(gamma), r2(beta), p1, p2,
                                cvec, use_bn=(i > 0))

        t = btab[src] + edge_attr @ w4
        s1 = jax.ops.segment_sum(t, dst, num_segments=N)
        s2 = jax.ops.segment_sum(t * t, dst, num_segments=N)
        tn = jax.ops.segment_min(t, dst, num_segments=N)
        tx = jax.ops.segment_max(t, dst, num_segments=N)
        tn = jnp.where(jnp.isfinite(tn), tn, 0.0)
        tx = jnp.where(jnp.isfinite(tx), tx, 0.0)

        y, cs, cq = _tc_post(xhat, a, s1, s2, tn, tx, cnt, q0, q1, q2, q3,
                             postb.reshape(1, D), linW, linb.reshape(1, D))
        xcur = y

    xfin = _tc_norm(xcur, cs, cq, r2(layers[3][8]), r2(layers[3][9]))

    bstart = jnp.searchsorted(batch.astype(jnp.int32),
                              jnp.arange(G + 1, dtype=jnp.int32))
    cg = (bstart[1:] - bstart[:-1]).astype(jnp.float32)[:, None]
    xa = jax.ops.segment_max(xfin, batch, num_segments=G)
    xs = jax.ops.segment_sum(xfin, batch, num_segments=G)

    w3p = jnp.pad(lin3_W, ((0, 0), (0, D - 2)))
    b3p = jnp.pad(lin3_b, (0, D - 2)).reshape(1, D)
    out = _tc_mlp(xa, xs, cg, lin1_W, lin1_b.reshape(1, D), lin2_W,
                  lin2_b.reshape(1, 64), w3p, b3p)
    return out[:, :2]


# speedup vs baseline: 2.6945x; 2.1975x over previous
import functools
import math

import jax
import jax.numpy as jnp
from jax import lax
from jax.experimental import pallas as pl
from jax.experimental.pallas import tpu as pltpu
from jax.experimental.pallas import tpu_sc as plsc

N = 10000
E = 160000
D = 128
G = 64
AVG_LOG = math.log(17.0)
BLK = 1000

NC = 2
NS = 16
NW = NC * NS
NP = 10240
NPW = NP // NW
GSZ = 64
NGRP = NPW // GSZ
C = 256
EP = E + 2 * C
RP = NP + 16
BIG = 3.0e38


def _sc_stats_body(btab, srcp, sdstp, eap, rs, w4, zer,
                   s1o, s2o, tno, txo,
                   rs_v, src_v, sdst_v, ea_v, brow_v, w4_v,
                   st1, st2, stn, stx, sem):
    wid = lax.axis_index("s") * NC + lax.axis_index("c")
    n0 = pl.multiple_of(wid * NPW, NPW)
    pltpu.sync_copy(rs.at[pl.ds(n0, NPW + 16)], rs_v)
    pltpu.sync_copy(w4, w4_v)
    pltpu.sync_copy(zer, st1)
    pltpu.sync_copy(zer, st2)
    pltpu.sync_copy(zer, stn)
    pltpu.sync_copy(zer, stx)
    e0 = rs_v[pl.ds(0, 16)][0]
    e_end = rs_v[pl.ds(NPW, 16)][0]
    eb0 = (e0 // 8) * 8
    nch = (e_end - eb0 + C - 1) // C

    zv = jnp.zeros((16,), jnp.float32)
    pv = jnp.full((16,), BIG, jnp.float32)
    nv = jnp.full((16,), -BIG, jnp.float32)
    carry0 = (jnp.int32(-1), n0,
              (zv,) * 8, (zv,) * 8, (pv,) * 8, (nv,) * 8)

    def flush(prev, gabs, s1a, s2a, tna, txa):
        row = (prev - gabs) * D
        for c in range(8):
            sl = pl.ds(row + c * 16, 16)
            st1[sl] = s1a[c]
            st2[sl] = s2a[c]
            stn[sl] = tna[c]
            stx[sl] = txa[c]

    def group_out(g):
        ga = pl.multiple_of(g * D, GSZ * D)
        pltpu.sync_copy(st1, s1o.at[pl.ds(ga, GSZ * D)])
        pltpu.sync_copy(st2, s2o.at[pl.ds(ga, GSZ * D)])
        pltpu.sync_copy(stn, tno.at[pl.ds(ga, GSZ * D)])
        pltpu.sync_copy(stx, txo.at[pl.ds(ga, GSZ * D)])
        pltpu.sync_copy(zer, st1)
        pltpu.sync_copy(zer, st2)
        pltpu.sync_copy(zer, stn)
        pltpu.sync_copy(zer, stx)

    def chunk_body(ci, carry):
        base = pl.multiple_of(eb0 + ci * C, 8)
        pltpu.sync_copy(srcp.at[pl.ds(base, C)], src_v)
        pltpu.sync_copy(sdstp.at[pl.ds(base, C)], sdst_v.at[pl.ds(0, C)])
        pltpu.sync_copy(eap.at[pl.ds(pl.multiple_of(4 * base, 8), 4 * C)],
                        ea_v.at[pl.ds(0, 4 * C)])
        pltpu.async_copy(btab.at[src_v], brow_v, sem).wait()

        def edge_body(e, car):
            prev, gabs, s1a, s2a, tna, txa = car
            d = sdst_v[pl.ds(e, 16)][0]
            in_rng = jnp.logical_and(d >= n0, d < n0 + NPW)
            is_new = jnp.logical_and(in_rng, d != prev)

            @pl.when(jnp.logical_and(is_new, prev >= 0))
            def _():
                flush(prev, gabs, s1a, s2a, tna, txa)

            d_cl = jnp.where(is_new, d, gabs)
            nadv = jnp.maximum((d_cl - gabs) // GSZ, 0)

            @pl.loop(0, nadv)
            def _(j):
                group_out(gabs + j * GSZ)

            gabs2 = gabs + nadv * GSZ

            eav = ea_v[pl.ds(4 * e, 16)]
            ea0 = eav[0]
            ea1 = eav[1]
            ea2 = eav[2]
            ea3 = eav[3]
            ns1, ns2, ntn, ntx = [], [], [], []
            for c in range(8):
                sl = pl.ds(c * 16, 16)
                t = (brow_v[e, sl]
                     + ea0 * w4_v[pl.ds(c * 16, 16)]
                     + ea1 * w4_v[pl.ds(D + c * 16, 16)]
                     + ea2 * w4_v[pl.ds(2 * D + c * 16, 16)]
                     + ea3 * w4_v[pl.ds(3 * D + c * 16, 16)])
                tt = t * t
                ns1.append(jnp.where(in_rng,
                                     jnp.where(is_new, t, s1a[c] + t),
                                     s1a[c]))
                ns2.append(jnp.where(in_rng,
                                     jnp.where(is_new, tt, s2a[c] + tt),
                                     s2a[c]))
                ntn.append(jnp.where(in_rng,
                                     jnp.where(is_new, t,
                                               jnp.minimum(tna[c], t)),
                                     tna[c]))
                ntx.append(jnp.where(in_rng,
                                     jnp.where(is_new, t,
                                               jnp.maximum(txa[c], t)),
                                     txa[c]))
            prev2 = jnp.where(is_new, d, prev)
            return (prev2, gabs2, tuple(ns1), tuple(ns2), tuple(ntn),
                    tuple(ntx))

        return lax.fori_loop(0, C, edge_body, carry)

    prev, gabs, s1a, s2a, tna, txa = lax.fori_loop(0, nch, chunk_body,
                                                   carry0)

    @pl.when(prev >= 0)
    def _():
        flush(prev, gabs, s1a, s2a, tna, txa)

    nfin = (n0 + NPW - gabs) // GSZ

    @pl.loop(0, nfin)
    def _(j):
        group_out(gabs + j * GSZ)


def _sc_stats(btab, srcp, sdstp, eap, rs, w4, zer):
    mesh = plsc.VectorSubcoreMesh(core_axis_name="c", subcore_axis_name="s")
    fn = functools.partial(
        pl.kernel, mesh=mesh,
        out_type=[jax.ShapeDtypeStruct((NP * D,), jnp.float32)] * 4,
        scratch_types=[
            pltpu.VMEM((NPW + 16,), jnp.int32),
            pltpu.VMEM((C,), jnp.int32),
            pltpu.VMEM((C + 16,), jnp.int32),
            pltpu.VMEM((4 * C + 16,), jnp.float32),
            pltpu.VMEM((C, D), jnp.float32),
            pltpu.VMEM((4 * D,), jnp.float32),
            pltpu.VMEM((GSZ * D,), jnp.float32),
            pltpu.VMEM((GSZ * D,), jnp.float32),
            pltpu.VMEM((GSZ * D,), jnp.float32),
            pltpu.VMEM((GSZ * D,), jnp.float32),
            pltpu.SemaphoreType.DMA,
        ],
    )(_sc_stats_body)
    return fn(btab, srcp, sdstp, eap, rs, w4, zer)


def _sc_readout_body(xf, bs, xo, bs_v, buf_v, row_v, sem):
    wid = lax.axis_index("s") * NC + lax.axis_index("c")
    pltpu.sync_copy(bs, bs_v)
    nv = jnp.full((16,), -BIG, jnp.float32)
    zv = jnp.zeros((16,), jnp.float32)
    for k in range(2):
        g = wid * 2 + k
        bv = bs_v[pl.ds(g, 16)]
        s = bv[0]
        e = bv[1]

        rb0 = (s // GSZ) * GSZ
        nck = jnp.maximum((e - rb0 + GSZ - 1) // GSZ, 0)

        @pl.loop(0, nck, init_carry=((nv,) * 8, (zv,) * 8))
        def seg_chunk(j, carry):
            mxa, sma = carry
            rb = rb0 + j * GSZ
            pltpu.sync_copy(
                xf.at[pl.ds(pl.multiple_of(rb * D, 8 * D), GSZ * D)], buf_v)
            r0 = jnp.maximum(s - rb, 0)
            rcount = jnp.minimum(e - rb, GSZ)

            def row_body(r, car):
                mx, sm = car
                nmx, nsm = [], []
                for c in range(8):
                    v = buf_v[pl.ds(r * D + c * 16, 16)]
                    nmx.append(jnp.maximum(mx[c], v))
                    nsm.append(sm[c] + v)
                return (tuple(nmx), tuple(nsm))

            return lax.fori_loop(r0, rcount, row_body, (mxa, sma))

        mxa, sma = seg_chunk
        for c in range(8):
            row_v[pl.ds(c * 16, 16)] = mxa[c]
            row_v[pl.ds(D + c * 16, 16)] = sma[c]
        pltpu.sync_copy(row_v,
                        xo.at[pl.ds(pl.multiple_of(g * 2 * D, 2 * D),
                                    2 * D)])


def _sc_readout(xf, bs):
    mesh = plsc.VectorSubcoreMesh(core_axis_name="c", subcore_axis_name="s")
    fn = functools.partial(
        pl.kernel, mesh=mesh,
        out_type=jax.ShapeDtypeStruct((G * 2 * D,), jnp.float32),
        scratch_types=[
            pltpu.VMEM((80,), jnp.int32),
            pltpu.VMEM((GSZ * D,), jnp.float32),
            pltpu.VMEM((2 * D,), jnp.float32),
            pltpu.SemaphoreType.DMA,
        ],
    )(_sc_readout_body)
    return fn(xf, bs)


def _pre_body(use_bn, x_ref, cs_ref, cq_ref, g_ref, bb_ref, p1_ref, p2_ref,
              cvec_ref, xhat_ref, a_ref, bt_ref):
    xv = x_ref[...]
    if use_bn:
        m = cs_ref[...] * (1.0 / N)
        v = cq_ref[...] * (1.0 / N) - m * m
        xv = (xv - m) / jnp.sqrt(v + 1e-5) * g_ref[...] + bb_ref[...]
        xv = jnp.maximum(xv, 0.0)
    xhat_ref[...] = xv
    a_ref[...] = jnp.dot(xv, p1_ref[...], preferred_element_type=jnp.float32)
    bt_ref[...] = (jnp.dot(xv, p2_ref[...], preferred_element_type=jnp.float32)
                   + cvec_ref[...])


def _tc_pre(x, cs, cq, gamma, beta, p1, p2, cvec, use_bn):
    row = lambda i: (i, 0)
    full = lambda i: (0, 0)
    return pl.pallas_call(
        functools.partial(_pre_body, use_bn),
        grid=(N // BLK,),
        in_specs=[
            pl.BlockSpec((BLK, D), row),
            pl.BlockSpec((1, D), full),
            pl.BlockSpec((1, D), full),
            pl.BlockSpec((1, D), full),
            pl.BlockSpec((1, D), full),
            pl.BlockSpec((D, D), full),
            pl.BlockSpec((D, D), full),
            pl.BlockSpec((1, D), full),
        ],
        out_specs=[
            pl.BlockSpec((BLK, D), row),
            pl.BlockSpec((BLK, D), row),
            pl.BlockSpec((BLK, D), row),
        ],
        out_shape=[jax.ShapeDtypeStruct((N, D), jnp.float32)] * 3,
    )(x, cs, cq, gamma, beta, p1, p2, cvec)


def _post_body(x_ref, a_ref, s1_ref, s2_ref, tn_ref, tx_ref, cnt_ref,
               q0_ref, q1_ref, q2_ref, q3_ref, pb_ref, lw_ref, lb_ref,
               y_ref, cs_ref, cq_ref):
    i = pl.program_id(0)
    a = a_ref[...]
    s1 = s1_ref[...]
    s2 = s2_ref[...]
    cnt = cnt_ref[...]
    cntc = jnp.maximum(cnt, 1.0)
    inv = 1.0 / cntc
    mean = (cnt * a + s1) * inv
    s2m = (cnt * a * a + 2.0 * a * s1 + s2) * inv
    var = s2m - mean * mean
    std = jnp.sqrt(jnp.maximum(var, 1e-5))
    has = cnt > 0.0
    mn = jnp.where(has, a + tn_ref[...], 0.0)
    mx = jnp.where(has, a + tx_ref[...], 0.0)
    A = jnp.concatenate([mean, mn, mx, std], axis=-1)
    lg = jnp.log(cntc + 1.0)
    amp = lg * (1.0 / AVG_LOG)
    att = AVG_LOG / lg
    dot = lambda u, w: jnp.dot(u, w[...], preferred_element_type=jnp.float32)
    z = (dot(x_ref[...], q0_ref) + dot(A, q1_ref) + dot(A * amp, q2_ref)
         + dot(A * att, q3_ref) + pb_ref[...])
    y = dot(z, lw_ref) + lb_ref[...]
    y_ref[...] = y

    @pl.when(i == 0)
    def _():
        cs_ref[...] = jnp.zeros_like(cs_ref)
        cq_ref[...] = jnp.zeros_like(cq_ref)

    cs_ref[...] += jnp.sum(y, axis=0, keepdims=True)
    cq_ref[...] += jnp.sum(y * y, axis=0, keepdims=True)


def _tc_post(x, a, s1, s2, tn, tx, cnt, q0, q1, q2, q3, pb, lw, lb):
    row = lambda i: (i, 0)
    full = lambda i: (0, 0)
    return pl.pallas_call(
        _post_body,
        grid=(N // BLK,),
        in_specs=[
            pl.BlockSpec((BLK, D), row),
            pl.BlockSpec((BLK, D), row),
            pl.BlockSpec((BLK, D), row),
            pl.BlockSpec((BLK, D), row),
            pl.BlockSpec((BLK, D), row),
            pl.BlockSpec((BLK, D), row),
            pl.BlockSpec((BLK, 1), row),
            pl.BlockSpec((D, D), full),
            pl.BlockSpec((4 * D, D), full),
            pl.BlockSpec((4 * D, D), full),
            pl.BlockSpec((4 * D, D), full),
            pl.BlockSpec((1, D), full),
            pl.BlockSpec((D, D), full),
            pl.BlockSpec((1, D), full),
        ],
        out_specs=[
            pl.BlockSpec((BLK, D), row),
            pl.BlockSpec((1, D), full),
            pl.BlockSpec((1, D), full),
        ],
        out_shape=[
            jax.ShapeDtypeStruct((N, D), jnp.float32),
            jax.ShapeDtypeStruct((1, D), jnp.float32),
            jax.ShapeDtypeStruct((1, D), jnp.float32),
        ],
    )(x, a, s1, s2, tn, tx, cnt, q0, q1, q2, q3, pb, lw, lb)


def _norm_body(x_ref, cs_ref, cq_ref, g_ref, bb_ref, o_ref):
    m = cs_ref[...] * (1.0 / N)
    v = cq_ref[...] * (1.0 / N) - m * m
    o_ref[...] = jnp.maximum(
        (x_ref[...] - m) / jnp.sqrt(v + 1e-5) * g_ref[...] + bb_ref[...], 0.0)


def _tc_norm(x, cs, cq, gamma, beta):
    row = lambda i: (i, 0)
    full = lambda i: (0, 0)
    return pl.pallas_call(
        _norm_body,
        grid=(N // BLK,),
        in_specs=[
            pl.BlockSpec((BLK, D), row),
            pl.BlockSpec((1, D), full),
            pl.BlockSpec((1, D), full),
            pl.BlockSpec((1, D), full),
            pl.BlockSpec((1, D), full),
        ],
        out_specs=pl.BlockSpec((BLK, D), row),
        out_shape=jax.ShapeDtypeStruct((N, D), jnp.float32),
    )(x, cs, cq, gamma, beta)


def _mlp_body(xa_ref, xs_ref, cg_ref, w1_ref, b1_ref, w2_ref, b2_ref,
              w3_ref, b3_ref, o_ref):
    cg = cg_ref[...]
    xa = jnp.where(cg > 0.0, xa_ref[...], 0.0)
    xb = xs_ref[...] / jnp.maximum(cg, 1.0)
    h = jnp.concatenate([xa, xb], axis=1)
    dot = lambda u, w: jnp.dot(u, w[...], preferred_element_type=jnp.float32)
    h = jnp.maximum(dot(h, w1_ref) + b1_ref[...], 0.0)
    h = jnp.maximum(dot(h, w2_ref) + b2_ref[...], 0.0)
    o_ref[...] = dot(h, w3_ref) + b3_ref[...]


def _tc_mlp(xa, xs, cg, w1, b1, w2, b2, w3p, b3p):
    return pl.pallas_call(
        _mlp_body,
        out_shape=jax.ShapeDtypeStruct((G, D), jnp.float32),
    )(xa, xs, cg, w1, b1, w2, b2, w3p, b3p)


def kernel(x, edge_index, edge_attr, intarna_energy, batch, covalent_edges,
           dropout_conv_1_2, dropout_conv_rest,
           c1_eW, c1_eb, c1_preW, c1_preb, c1_postW, c1_postb, c1_linW,
           c1_linb, c1_gamma, c1_beta,
           c2_eW, c2_eb, c2_preW, c2_preb, c2_postW, c2_postb, c2_linW,
           c2_linb, c2_gamma, c2_beta,
           c3_eW, c3_eb, c3_preW, c3_preb, c3_postW, c3_postb, c3_linW,
           c3_linb, c3_gamma, c3_beta,
           c4_eW, c4_eb, c4_preW, c4_preb, c4_postW, c4_postb, c4_linW,
           c4_linb, c4_gamma, c4_beta,
           lin1_W, lin1_b, lin2_W, lin2_b, lin3_W, lin3_b):
    src = edge_index[0]
    dst = edge_index[1]

    key = (dst.astype(jnp.uint32) << 18) | jnp.arange(E, dtype=jnp.uint32)
    skey = jax.lax.sort(key)
    perm = (skey & jnp.uint32((1 << 18) - 1)).astype(jnp.int32)
    sdst = (skey >> 18).astype(jnp.int32)
    row_start = jnp.searchsorted(sdst, jnp.arange(N + 1, dtype=jnp.int32))
    cnt = (row_start[1:] - row_start[:-1]).astype(jnp.float32)[:, None]

    srcp = jnp.concatenate([src[perm].astype(jnp.int32),
                            jnp.zeros((EP - E,), jnp.int32)])
    sdstp = jnp.concatenate([sdst, jnp.full((EP - E,), NP, jnp.int32)])
    eap = jnp.concatenate([edge_attr[perm],
                           jnp.zeros((EP - E, 4),
                                     jnp.float32)]).reshape(4 * EP)
    rsp = jnp.concatenate([
        row_start.astype(jnp.int32),
        jnp.full((RP - (N + 1),), E, jnp.int32)])
    zer = jnp.zeros((GSZ * D,), jnp.float32)

    layers = [
        (c1_eW, c1_eb, c1_preW, c1_preb, c1_postW, c1_postb, c1_linW,
         c1_linb, c1_gamma, c1_beta),
        (c2_eW, c2_eb, c2_preW, c2_preb, c2_postW, c2_postb, c2_linW,
         c2_linb, c2_gamma, c2_beta),
        (c3_eW, c3_eb, c3_preW, c3_preb, c3_postW, c3_postb, c3_linW,
         c3_linb, c3_gamma, c3_beta),
        (c4_eW, c4_eb, c4_preW, c4_preb, c4_postW, c4_postb, c4_linW,
         c4_linb, c4_gamma, c4_beta),
    ]

    r2 = lambda v: v.reshape(1, D)
    xcur = x
    cs = jnp.zeros((1, D), jnp.float32)
    cq = jnp.zeros((1, D), jnp.float32)
    for i, (eW, eb, preW, preb, postW, postb, linW, linb, gamma,
            beta) in enumerate(layers):
        p1 = preW[:D]
        p2 = preW[D:2 * D]
        p3 = preW[2 * D:]
        w4 = eW @ p3
        cvec = (eb @ p3 + preb).reshape(1, D)
        q0 = postW[:D]
        q1 = postW[D:5 * D]
        q2 = postW[5 * D:9 * D]
        q3 = postW[9 * D:]

        xhat, a, btab = _tc_pre(xcur, cs, cq, r2(gamma), r2(beta), p1, p2,
                                cvec, use_bn=(i > 0))

        s1, s2, tn, tx = _sc_stats(btab, srcp, sdstp, eap, rsp,
                                   w4.reshape(4 * D), zer)
        s1 = s1.reshape(NP, D)[:N]
        s2 = s2.reshape(NP, D)[:N]
        tn = tn.reshape(NP, D)[:N]
        tx = tx.reshape(NP, D)[:N]

        y, cs, cq = _tc_post(xhat, a, s1, s2, tn, tx, cnt, q0, q1, q2, q3,
                             postb.reshape(1, D), linW, linb.reshape(1, D))
        xcur = y

    xfin = _tc_norm(xcur, cs, cq, r2(layers[3][8]), r2(layers[3][9]))

    bstart = jnp.searchsorted(batch.astype(jnp.int32),
                              jnp.arange(G + 1, dtype=jnp.int32))
    cg = (bstart[1:] - bstart[:-1]).astype(jnp.float32)[:, None]
    bsp = jnp.concatenate([bstart.astype(jnp.int32),
                           jnp.full((80 - (G + 1),), N, jnp.int32)])
    xfp = jnp.concatenate([xfin, jnp.zeros((GSZ, D),
                                           jnp.float32)]).reshape(-1)
    xro = _sc_readout(xfp, bsp).reshape(G, 2, D)
    xa = xro[:, 0, :]
    xs = xro[:, 1, :]

    w3p = jnp.pad(lin3_W, ((0, 0), (0, D - 2)))
    b3p = jnp.pad(lin3_b, (0, D - 2)).reshape(1, D)
    out = _tc_mlp(xa, xs, cg, lin1_W, lin1_b.reshape(1, D), lin2_W,
                  lin2_b.reshape(1, 64), w3p, b3p)
    return out[:, :2]


# speedup vs baseline: 2.9680x; 1.1015x over previous
import functools
import math

import jax
import jax.numpy as jnp
from jax import lax
from jax.experimental import pallas as pl
from jax.experimental.pallas import tpu as pltpu
from jax.experimental.pallas import tpu_sc as plsc

N = 10000
E = 160000
D = 128
G = 64
AVG_LOG = math.log(17.0)
BLK = 1000

NC = 2
NS = 16
NW = NC * NS
NP = 10240
NPW = NP // NW
GSZ = 64
NGRP = NPW // GSZ
C = 256
EP = E + 2 * C
RP = NP + 16
BIG = 3.0e38


def _sc_stats_body(btab, srcp, sdstp, eap, rs, w4, zer,
                   s1o, s2o, tno, txo,
                   rs_v, src_v, sdst_v, ea_v, brow_v, w4_v,
                   st1, st2, stn, stx, sem):
    wid = lax.axis_index("s") * NC + lax.axis_index("c")
    n0 = pl.multiple_of(wid * NPW, NPW)
    pltpu.sync_copy(rs.at[pl.ds(n0, NPW + 16)], rs_v)
    pltpu.sync_copy(w4, w4_v)
    pltpu.sync_copy(zer, st1)
    pltpu.sync_copy(zer, st2)
    pltpu.sync_copy(zer, stn)
    pltpu.sync_copy(zer, stx)
    e0 = rs_v[pl.ds(0, 16)][0]
    e_end = rs_v[pl.ds(NPW, 16)][0]
    eb0 = (e0 // 8) * 8
    nch = (e_end - eb0 + C - 1) // C

    zv = jnp.zeros((16,), jnp.float32)
    pv = jnp.full((16,), BIG, jnp.float32)
    nv = jnp.full((16,), -BIG, jnp.float32)
    carry0 = (jnp.int32(-1), n0,
              (zv,) * 8, (zv,) * 8, (pv,) * 8, (nv,) * 8)

    def flush(prev, gabs, s1a, s2a, tna, txa):
        row = (prev - gabs) * D
        for c in range(8):
            sl = pl.ds(row + c * 16, 16)
            st1[sl] = s1a[c]
            st2[sl] = s2a[c]
            stn[sl] = tna[c]
            stx[sl] = txa[c]

    def group_out(g):
        ga = pl.multiple_of(g * D, GSZ * D)
        pltpu.sync_copy(st1, s1o.at[pl.ds(ga, GSZ * D)])
        pltpu.sync_copy(st2, s2o.at[pl.ds(ga, GSZ * D)])
        pltpu.sync_copy(stn, tno.at[pl.ds(ga, GSZ * D)])
        pltpu.sync_copy(stx, txo.at[pl.ds(ga, GSZ * D)])
        pltpu.sync_copy(zer, st1)
        pltpu.sync_copy(zer, st2)
        pltpu.sync_copy(zer, stn)
        pltpu.sync_copy(zer, stx)

    def chunk_body(ci, carry):
        base = pl.multiple_of(eb0 + ci * C, 8)
        pltpu.sync_copy(srcp.at[pl.ds(base, C)], src_v)
        pltpu.sync_copy(sdstp.at[pl.ds(base, C)], sdst_v.at[pl.ds(0, C)])
        pltpu.sync_copy(eap.at[pl.ds(pl.multiple_of(4 * base, 8), 4 * C)],
                        ea_v.at[pl.ds(0, 4 * C)])
        pltpu.async_copy(btab.at[src_v], brow_v, sem).wait()

        lo = jnp.maximum(e0 - base, 0)
        hi = jnp.minimum(e_end - base, C)

        def edge_body(e, car):
            prev, gabs, s1a, s2a, tna, txa = car
            d = sdst_v[pl.ds(e, 16)][0]
            is_new = d != prev

            @pl.when(jnp.logical_and(is_new, prev >= 0))
            def _():
                flush(prev, gabs, s1a, s2a, tna, txa)

            nadv = jnp.where(is_new, (d - gabs) // GSZ, 0)

            @pl.loop(0, nadv)
            def _(j):
                group_out(gabs + j * GSZ)

            gabs = gabs + nadv * GSZ

            eav = ea_v[pl.ds(4 * e, 16)]
            ea0 = eav[0]
            ea1 = eav[1]
            ea2 = eav[2]
            ea3 = eav[3]
            ns1, ns2, ntn, ntx = [], [], [], []
            for c in range(8):
                t = (brow_v[e, pl.ds(c * 16, 16)]
                     + ea0 * w4_v[pl.ds(c * 16, 16)]
                     + ea1 * w4_v[pl.ds(D + c * 16, 16)]
                     + ea2 * w4_v[pl.ds(2 * D + c * 16, 16)]
                     + ea3 * w4_v[pl.ds(3 * D + c * 16, 16)])
                ns1.append(jnp.where(is_new, zv, s1a[c]) + t)
                ns2.append(jnp.where(is_new, zv, s2a[c]) + t * t)
                ntn.append(jnp.minimum(jnp.where(is_new, pv, tna[c]), t))
                ntx.append(jnp.maximum(jnp.where(is_new, nv, txa[c]), t))
            return (d, gabs, tuple(ns1), tuple(ns2), tuple(ntn),
                    tuple(ntx))

        return lax.fori_loop(lo, hi, edge_body, carry)

    prev, gabs, s1a, s2a, tna, txa = lax.fori_loop(0, nch, chunk_body,
                                                   carry0)

    @pl.when(prev >= 0)
    def _():
        flush(prev, gabs, s1a, s2a, tna, txa)

    nfin = (n0 + NPW - gabs) // GSZ

    @pl.loop(0, nfin)
    def _(j):
        group_out(gabs + j * GSZ)


def _sc_stats(btab, srcp, sdstp, eap, rs, w4, zer):
    mesh = plsc.VectorSubcoreMesh(core_axis_name="c", subcore_axis_name="s")
    fn = functools.partial(
        pl.kernel, mesh=mesh,
        out_type=[jax.ShapeDtypeStruct((NP * D,), jnp.float32)] * 4,
        scratch_types=[
            pltpu.VMEM((NPW + 16,), jnp.int32),
            pltpu.VMEM((C,), jnp.int32),
            pltpu.VMEM((C + 16,), jnp.int32),
            pltpu.VMEM((4 * C + 16,), jnp.float32),
            pltpu.VMEM((C, D), jnp.float32),
            pltpu.VMEM((4 * D,), jnp.float32),
            pltpu.VMEM((GSZ * D,), jnp.float32),
            pltpu.VMEM((GSZ * D,), jnp.float32),
            pltpu.VMEM((GSZ * D,), jnp.float32),
            pltpu.VMEM((GSZ * D,), jnp.float32),
            pltpu.SemaphoreType.DMA,
        ],
    )(_sc_stats_body)
    return fn(btab, srcp, sdstp, eap, rs, w4, zer)


def _sc_readout_body(xf, bs, xo, bs_v, buf_v, row_v, sem):
    wid = lax.axis_index("s") * NC + lax.axis_index("c")
    pltpu.sync_copy(bs, bs_v)
    nv = jnp.full((16,), -BIG, jnp.float32)
    zv = jnp.zeros((16,), jnp.float32)
    for k in range(2):
        g = wid * 2 + k
        bv = bs_v[pl.ds(g, 16)]
        s = bv[0]
        e = bv[1]

        rb0 = (s // GSZ) * GSZ
        nck = jnp.maximum((e - rb0 + GSZ - 1) // GSZ, 0)

        @pl.loop(0, nck, init_carry=((nv,) * 8, (zv,) * 8))
        def seg_chunk(j, carry):
            mxa, sma = carry
            rb = rb0 + j * GSZ
            pltpu.sync_copy(
                xf.at[pl.ds(pl.multiple_of(rb * D, 8 * D), GSZ * D)], buf_v)
            r0 = jnp.maximum(s - rb, 0)
            rcount = jnp.minimum(e - rb, GSZ)

            def row_body(r, car):
                mx, sm = car
                nmx, nsm = [], []
                for c in range(8):
                    v = buf_v[pl.ds(r * D + c * 16, 16)]
                    nmx.append(jnp.maximum(mx[c], v))
                    nsm.append(sm[c] + v)
                return (tuple(nmx), tuple(nsm))

            return lax.fori_loop(r0, rcount, row_body, (mxa, sma))

        mxa, sma = seg_chunk
        for c in range(8):
            row_v[pl.ds(c * 16, 16)] = mxa[c]
            row_v[pl.ds(D + c * 16, 16)] = sma[c]
        pltpu.sync_copy(row_v,
                        xo.at[pl.ds(pl.multiple_of(g * 2 * D, 2 * D),
                                    2 * D)])


def _sc_readout(xf, bs):
    mesh = plsc.VectorSubcoreMesh(core_axis_name="c", subcore_axis_name="s")
    fn = functools.partial(
        pl.kernel, mesh=mesh,
        out_type=jax.ShapeDtypeStruct((G * 2 * D,), jnp.float32),
        scratch_types=[
            pltpu.VMEM((80,), jnp.int32),
            pltpu.VMEM((GSZ * D,), jnp.float32),
            pltpu.VMEM((2 * D,), jnp.float32),
            pltpu.SemaphoreType.DMA,
        ],
    )(_sc_readout_body)
    return fn(xf, bs)


def _pre_body(use_bn, x_ref, cs_ref, cq_ref, g_ref, bb_ref, p1_ref, p2_ref,
              cvec_ref, xhat_ref, a_ref, bt_ref):
    xv = x_ref[...]
    if use_bn:
        m = cs_ref[...] * (1.0 / N)
        v = cq_ref[...] * (1.0 / N) - m * m
        xv = (xv - m) / jnp.sqrt(v + 1e-5) * g_ref[...] + bb_ref[...]
        xv = jnp.maximum(xv, 0.0)
    xhat_ref[...] = xv
    a_ref[...] = jnp.dot(xv, p1_ref[...], preferred_element_type=jnp.float32)
    bt_ref[...] = (jnp.dot(xv, p2_ref[...], preferred_element_type=jnp.float32)
                   + cvec_ref[...])


def _tc_pre(x, cs, cq, gamma, beta, p1, p2, cvec, use_bn):
    row = lambda i: (i, 0)
    full = lambda i: (0, 0)
    return pl.pallas_call(
        functools.partial(_pre_body, use_bn),
        grid=(N // BLK,),
        in_specs=[
            pl.BlockSpec((BLK, D), row),
            pl.BlockSpec((1, D), full),
            pl.BlockSpec((1, D), full),
            pl.BlockSpec((1, D), full),
            pl.BlockSpec((1, D), full),
            pl.BlockSpec((D, D), full),
            pl.BlockSpec((D, D), full),
            pl.BlockSpec((1, D), full),
        ],
        out_specs=[
            pl.BlockSpec((BLK, D), row),
            pl.BlockSpec((BLK, D), row),
            pl.BlockSpec((BLK, D), row),
        ],
        out_shape=[jax.ShapeDtypeStruct((N, D), jnp.float32)] * 3,
    )(x, cs, cq, gamma, beta, p1, p2, cvec)


def _post_body(x_ref, a_ref, s1_ref, s2_ref, tn_ref, tx_ref, cnt_ref,
               q0_ref, q1_ref, q2_ref, q3_ref, pb_ref, lw_ref, lb_ref,
               y_ref, cs_ref, cq_ref):
    i = pl.program_id(0)
    a = a_ref[...]
    s1 = s1_ref[...]
    s2 = s2_ref[...]
    cnt = cnt_ref[...]
    cntc = jnp.maximum(cnt, 1.0)
    inv = 1.0 / cntc
    mean = (cnt * a + s1) * inv
    s2m = (cnt * a * a + 2.0 * a * s1 + s2) * inv
    var = s2m - mean * mean
    std = jnp.sqrt(jnp.maximum(var, 1e-5))
    has = cnt > 0.0
    mn = jnp.where(has, a + tn_ref[...], 0.0)
    mx = jnp.where(has, a + tx_ref[...], 0.0)
    A = jnp.concatenate([mean, mn, mx, std], axis=-1)
    lg = jnp.log(cntc + 1.0)
    amp = lg * (1.0 / AVG_LOG)
    att = AVG_LOG / lg
    dot = lambda u, w: jnp.dot(u, w[...], preferred_element_type=jnp.float32)
    z = (dot(x_ref[...], q0_ref) + dot(A, q1_ref) + dot(A * amp, q2_ref)
         + dot(A * att, q3_ref) + pb_ref[...])
    y = dot(z, lw_ref) + lb_ref[...]
    y_ref[...] = y

    @pl.when(i == 0)
    def _():
        cs_ref[...] = jnp.zeros_like(cs_ref)
        cq_ref[...] = jnp.zeros_like(cq_ref)

    cs_ref[...] += jnp.sum(y, axis=0, keepdims=True)
    cq_ref[...] += jnp.sum(y * y, axis=0, keepdims=True)


def _tc_post(x, a, s1, s2, tn, tx, cnt, q0, q1, q2, q3, pb, lw, lb):
    row = lambda i: (i, 0)
    full = lambda i: (0, 0)
    return pl.pallas_call(
        _post_body,
        grid=(N // BLK,),
        in_specs=[
            pl.BlockSpec((BLK, D), row),
            pl.BlockSpec((BLK, D), row),
            pl.BlockSpec((BLK, D), row),
            pl.BlockSpec((BLK, D), row),
            pl.BlockSpec((BLK, D), row),
            pl.BlockSpec((BLK, D), row),
            pl.BlockSpec((BLK, 1), row),
            pl.BlockSpec((D, D), full),
            pl.BlockSpec((4 * D, D), full),
            pl.BlockSpec((4 * D, D), full),
            pl.BlockSpec((4 * D, D), full),
            pl.BlockSpec((1, D), full),
            pl.BlockSpec((D, D), full),
            pl.BlockSpec((1, D), full),
        ],
        out_specs=[
            pl.BlockSpec((BLK, D), row),
            pl.BlockSpec((1, D), full),
            pl.BlockSpec((1, D), full),
        ],
        out_shape=[
            jax.ShapeDtypeStruct((N, D), jnp.float32),
            jax.ShapeDtypeStruct((1, D), jnp.float32),
            jax.ShapeDtypeStruct((1, D), jnp.float32),
        ],
    )(x, a, s1, s2, tn, tx, cnt, q0, q1, q2, q3, pb, lw, lb)


def _norm_body(x_ref, cs_ref, cq_ref, g_ref, bb_ref, o_ref):
    m = cs_ref[...] * (1.0 / N)
    v = cq_ref[...] * (1.0 / N) - m * m
    o_ref[...] = jnp.maximum(
        (x_ref[...] - m) / jnp.sqrt(v + 1e-5) * g_ref[...] + bb_ref[...], 0.0)


def _tc_norm(x, cs, cq, gamma, beta):
    row = lambda i: (i, 0)
    full = lambda i: (0, 0)
    return pl.pallas_call(
        _norm_body,
        grid=(N // BLK,),
        in_specs=[
            pl.BlockSpec((BLK, D), row),
            pl.BlockSpec((1, D), full),
            pl.BlockSpec((1, D), full),
            pl.BlockSpec((1, D), full),
            pl.BlockSpec((1, D), full),
        ],
        out_specs=pl.BlockSpec((BLK, D), row),
        out_shape=jax.ShapeDtypeStruct((N, D), jnp.float32),
    )(x, cs, cq, gamma, beta)


def _mlp_body(xa_ref, xs_ref, cg_ref, w1_ref, b1_ref, w2_ref, b2_ref,
              w3_ref, b3_ref, o_ref):
    cg = cg_ref[...]
    xa = jnp.where(cg > 0.0, xa_ref[...], 0.0)
    xb = xs_ref[...] / jnp.maximum(cg, 1.0)
    h = jnp.concatenate([xa, xb], axis=1)
    dot = lambda u, w: jnp.dot(u, w[...], preferred_element_type=jnp.float32)
    h = jnp.maximum(dot(h, w1_ref) + b1_ref[...], 0.0)
    h = jnp.maximum(dot(h, w2_ref) + b2_ref[...], 0.0)
    o_ref[...] = dot(h, w3_ref) + b3_ref[...]


def _tc_mlp(xa, xs, cg, w1, b1, w2, b2, w3p, b3p):
    return pl.pallas_call(
        _mlp_body,
        out_shape=jax.ShapeDtypeStruct((G, D), jnp.float32),
    )(xa, xs, cg, w1, b1, w2, b2, w3p, b3p)


def kernel(x, edge_index, edge_attr, intarna_energy, batch, covalent_edges,
           dropout_conv_1_2, dropout_conv_rest,
           c1_eW, c1_eb, c1_preW, c1_preb, c1_postW, c1_postb, c1_linW,
           c1_linb, c1_gamma, c1_beta,
           c2_eW, c2_eb, c2_preW, c2_preb, c2_postW, c2_postb, c2_linW,
           c2_linb, c2_gamma, c2_beta,
           c3_eW, c3_eb, c3_preW, c3_preb, c3_postW, c3_postb, c3_linW,
           c3_linb, c3_gamma, c3_beta,
           c4_eW, c4_eb, c4_preW, c4_preb, c4_postW, c4_postb, c4_linW,
           c4_linb, c4_gamma, c4_beta,
           lin1_W, lin1_b, lin2_W, lin2_b, lin3_W, lin3_b):
    src = edge_index[0]
    dst = edge_index[1]

    key = (dst.astype(jnp.uint32) << 18) | jnp.arange(E, dtype=jnp.uint32)
    skey = jax.lax.sort(key)
    perm = (skey & jnp.uint32((1 << 18) - 1)).astype(jnp.int32)
    sdst = (skey >> 18).astype(jnp.int32)
    row_start = jnp.searchsorted(sdst, jnp.arange(N + 1, dtype=jnp.int32))
    cnt = (row_start[1:] - row_start[:-1]).astype(jnp.float32)[:, None]

    srcp = jnp.concatenate([src[perm].astype(jnp.int32),
                            jnp.zeros((EP - E,), jnp.int32)])
    sdstp = jnp.concatenate([sdst, jnp.full((EP - E,), NP, jnp.int32)])
    eap = jnp.concatenate([edge_attr[perm],
                           jnp.zeros((EP - E, 4),
                                     jnp.float32)]).reshape(4 * EP)
    rsp = jnp.concatenate([
        row_start.astype(jnp.int32),
        jnp.full((RP - (N + 1),), E, jnp.int32)])
    zer = jnp.zeros((GSZ * D,), jnp.float32)

    layers = [
        (c1_eW, c1_eb, c1_preW, c1_preb, c1_postW, c1_postb, c1_linW,
         c1_linb, c1_gamma, c1_beta),
        (c2_eW, c2_eb, c2_preW, c2_preb, c2_postW, c2_postb, c2_linW,
         c2_linb, c2_gamma, c2_beta),
        (c3_eW, c3_eb, c3_preW, c3_preb, c3_postW, c3_postb, c3_linW,
         c3_linb, c3_gamma, c3_beta),
        (c4_eW, c4_eb, c4_preW, c4_preb, c4_postW, c4_postb, c4_linW,
         c4_linb, c4_gamma, c4_beta),
    ]

    r2 = lambda v: v.reshape(1, D)
    xcur = x
    cs = jnp.zeros((1, D), jnp.float32)
    cq = jnp.zeros((1, D), jnp.float32)
    for i, (eW, eb, preW, preb, postW, postb, linW, linb, gamma,
            beta) in enumerate(layers):
        p1 = preW[:D]
        p2 = preW[D:2 * D]
        p3 = preW[2 * D:]
        w4 = eW @ p3
        cvec = (eb @ p3 + preb).reshape(1, D)
        q0 = postW[:D]
        q1 = postW[D:5 * D]
        q2 = postW[5 * D:9 * D]
        q3 = postW[9 * D:]

        xhat, a, btab = _tc_pre(xcur, cs, cq, r2(gamma), r2(beta), p1, p2,
                                cvec, use_bn=(i > 0))

        s1, s2, tn, tx = _sc_stats(btab, srcp, sdstp, eap, rsp,
                                   w4.reshape(4 * D), zer)
        s1 = s1.reshape(NP, D)[:N]
        s2 = s2.reshape(NP, D)[:N]
        tn = tn.reshape(NP, D)[:N]
        tx = tx.reshape(NP, D)[:N]

        y, cs, cq = _tc_post(xhat, a, s1, s2, tn, tx, cnt, q0, q1, q2, q3,
                             postb.reshape(1, D), linW, linb.reshape(1, D))
        xcur = y

    xfin = _tc_norm(xcur, cs, cq, r2(layers[3][8]), r2(layers[3][9]))

    bstart = jnp.searchsorted(batch.astype(jnp.int32),
                              jnp.arange(G + 1, dtype=jnp.int32))
    cg = (bstart[1:] - bstart[:-1]).astype(jnp.float32)[:, None]
    bsp = jnp.concatenate([bstart.astype(jnp.int32),
                           jnp.full((80 - (G + 1),), N, jnp.int32)])
    xfp = jnp.concatenate([xfin, jnp.zeros((GSZ, D),
                                           jnp.float32)]).reshape(-1)
    xro = _sc_readout(xfp, bsp).reshape(G, 2, D)
    xa = xro[:, 0, :]
    xs = xro[:, 1, :]

    w3p = jnp.pad(lin3_W, ((0, 0), (0, D - 2)))
    b3p = jnp.pad(lin3_b, (0, D - 2)).reshape(1, D)
    out = _tc_mlp(xa, xs, cg, lin1_W, lin1_b.reshape(1, D), lin2_W,
                  lin2_b.reshape(1, 64), w3p, b3p)
    return out[:, :2]


# speedup vs baseline: 3.0322x; 1.0216x over previous
import functools
import math

import jax
import jax.numpy as jnp
from jax import lax
from jax.experimental import pallas as pl
from jax.experimental.pallas import tpu as pltpu
from jax.experimental.pallas import tpu_sc as plsc

N = 10000
E = 160000
D = 128
G = 64
AVG_LOG = math.log(17.0)
BLK = 1000

NC = 2
NS = 16
NW = NC * NS
NP = 10240
NPW = NP // NW
GSZ = 64
NGRP = NPW // GSZ
C = 256
EP = E + 2 * C
RP = NP + 16
BIG = 3.0e38


def _sc_stats_body(btab, srcp, sdstp, eap, rs, w4, zer,
                   s1o, s2o, tno, txo,
                   rs_v, src_v, sdst_v, ea_v, brow_v, w4_v,
                   st1, st2, stn, stx, sem):
    wid = lax.axis_index("s") * NC + lax.axis_index("c")
    n0 = pl.multiple_of(wid * NPW, NPW)
    pltpu.sync_copy(rs.at[pl.ds(n0, NPW + 16)], rs_v)
    pltpu.sync_copy(w4, w4_v)
    pltpu.sync_copy(zer, st1)
    pltpu.sync_copy(zer, st2)
    pltpu.sync_copy(zer, stn)
    pltpu.sync_copy(zer, stx)
    e0 = rs_v[pl.ds(0, 16)][0]
    e_end = rs_v[pl.ds(NPW, 16)][0]
    eb0 = (e0 // 8) * 8
    nch = (e_end - eb0 + C - 1) // C

    zv = jnp.zeros((16,), jnp.float32)
    pv = jnp.full((16,), BIG, jnp.float32)
    nv = jnp.full((16,), -BIG, jnp.float32)
    carry0 = (jnp.int32(-1), n0,
              (zv,) * 8, (zv,) * 8, (pv,) * 8, (nv,) * 8)

    def flush(prev, gabs, s1a, s2a, tna, txa):
        row = (prev - gabs) * D
        for c in range(8):
            sl = pl.ds(row + c * 16, 16)
            st1[sl] = s1a[c]
            st2[sl] = s2a[c]
            stn[sl] = tna[c]
            stx[sl] = txa[c]

    def group_out(g):
        ga = pl.multiple_of(g * D, GSZ * D)
        pltpu.sync_copy(st1, s1o.at[pl.ds(ga, GSZ * D)])
        pltpu.sync_copy(st2, s2o.at[pl.ds(ga, GSZ * D)])
        pltpu.sync_copy(stn, tno.at[pl.ds(ga, GSZ * D)])
        pltpu.sync_copy(stx, txo.at[pl.ds(ga, GSZ * D)])
        pltpu.sync_copy(zer, st1)
        pltpu.sync_copy(zer, st2)
        pltpu.sync_copy(zer, stn)
        pltpu.sync_copy(zer, stx)

    def chunk_body(ci, carry):
        base = pl.multiple_of(eb0 + ci * C, 8)
        cp1 = pltpu.async_copy(srcp.at[pl.ds(base, C)], src_v, sem)
        cp2 = pltpu.async_copy(sdstp.at[pl.ds(base, C)],
                               sdst_v.at[pl.ds(0, C)], sem)
        cp3 = pltpu.async_copy(
            eap.at[pl.ds(pl.multiple_of(4 * base, 8), 4 * C)],
            ea_v.at[pl.ds(0, 4 * C)], sem)
        cp1.wait()
        cp2.wait()
        cp3.wait()
        pltpu.async_copy(btab.at[src_v], brow_v, sem).wait()

        lo = jnp.maximum(e0 - base, 0)
        hi = jnp.minimum(e_end - base, C)

        def edge_body(e, car):
            prev, gabs, s1a, s2a, tna, txa = car
            d = sdst_v[pl.ds(e, 16)][0]
            is_new = d != prev

            @pl.when(jnp.logical_and(is_new, prev >= 0))
            def _():
                flush(prev, gabs, s1a, s2a, tna, txa)

            nadv = jnp.where(is_new, (d - gabs) // GSZ, 0)

            @pl.when(nadv > 0)
            def _():
                @pl.loop(0, nadv)
                def _(j):
                    group_out(gabs + j * GSZ)

            gabs = gabs + nadv * GSZ

            eav = ea_v[pl.ds(4 * e, 16)]
            ea0 = eav[0]
            ea1 = eav[1]
            ea2 = eav[2]
            ea3 = eav[3]
            ns1, ns2, ntn, ntx = [], [], [], []
            for c in range(8):
                t = (brow_v[e, pl.ds(c * 16, 16)]
                     + ea0 * w4_v[pl.ds(c * 16, 16)]
                     + ea1 * w4_v[pl.ds(D + c * 16, 16)]
                     + ea2 * w4_v[pl.ds(2 * D + c * 16, 16)]
                     + ea3 * w4_v[pl.ds(3 * D + c * 16, 16)])
                ns1.append(jnp.where(is_new, zv, s1a[c]) + t)
                ns2.append(jnp.where(is_new, zv, s2a[c]) + t * t)
                ntn.append(jnp.minimum(jnp.where(is_new, pv, tna[c]), t))
                ntx.append(jnp.maximum(jnp.where(is_new, nv, txa[c]), t))
            return (d, gabs, tuple(ns1), tuple(ns2), tuple(ntn),
                    tuple(ntx))

        return lax.fori_loop(lo, hi, edge_body, carry)

    prev, gabs, s1a, s2a, tna, txa = lax.fori_loop(0, nch, chunk_body,
                                                   carry0)

    @pl.when(prev >= 0)
    def _():
        flush(prev, gabs, s1a, s2a, tna, txa)

    nfin = (n0 + NPW - gabs) // GSZ

    @pl.loop(0, nfin)
    def _(j):
        group_out(gabs + j * GSZ)


def _sc_stats(btab, srcp, sdstp, eap, rs, w4, zer):
    mesh = plsc.VectorSubcoreMesh(core_axis_name="c", subcore_axis_name="s")
    fn = functools.partial(
        pl.kernel, mesh=mesh,
        out_type=[jax.ShapeDtypeStruct((NP * D,), jnp.float32)] * 4,
        scratch_types=[
            pltpu.VMEM((NPW + 16,), jnp.int32),
            pltpu.VMEM((C,), jnp.int32),
            pltpu.VMEM((C + 16,), jnp.int32),
            pltpu.VMEM((4 * C + 16,), jnp.float32),
            pltpu.VMEM((C, D), jnp.float32),
            pltpu.VMEM((4 * D,), jnp.float32),
            pltpu.VMEM((GSZ * D,), jnp.float32),
            pltpu.VMEM((GSZ * D,), jnp.float32),
            pltpu.VMEM((GSZ * D,), jnp.float32),
            pltpu.VMEM((GSZ * D,), jnp.float32),
            pltpu.SemaphoreType.DMA,
        ],
    )(_sc_stats_body)
    return fn(btab, srcp, sdstp, eap, rs, w4, zer)


def _sc_readout_body(xf, bs, xo, bs_v, buf_v, row_v, sem):
    wid = lax.axis_index("s") * NC + lax.axis_index("c")
    pltpu.sync_copy(bs, bs_v)
    nv = jnp.full((16,), -BIG, jnp.float32)
    zv = jnp.zeros((16,), jnp.float32)
    for k in range(2):
        g = wid * 2 + k
        bv = bs_v[pl.ds(g, 16)]
        s = bv[0]
        e = bv[1]

        rb0 = (s // GSZ) * GSZ
        nck = jnp.maximum((e - rb0 + GSZ - 1) // GSZ, 0)

        @pl.loop(0, nck, init_carry=((nv,) * 8, (zv,) * 8))
        def seg_chunk(j, carry):
            mxa, sma = carry
            rb = rb0 + j * GSZ
            pltpu.sync_copy(
                xf.at[pl.ds(pl.multiple_of(rb * D, 8 * D), GSZ * D)], buf_v)
            r0 = jnp.maximum(s - rb, 0)
            rcount = jnp.minimum(e - rb, GSZ)

            def row_body(r, car):
                mx, sm = car
                nmx, nsm = [], []
                for c in range(8):
                    v = buf_v[pl.ds(r * D + c * 16, 16)]
                    nmx.append(jnp.maximum(mx[c], v))
                    nsm.append(sm[c] + v)
                return (tuple(nmx), tuple(nsm))

            return lax.fori_loop(r0, rcount, row_body, (mxa, sma))

        mxa, sma = seg_chunk
        for c in range(8):
            row_v[pl.ds(c * 16, 16)] = mxa[c]
            row_v[pl.ds(D + c * 16, 16)] = sma[c]
        pltpu.sync_copy(row_v,
                        xo.at[pl.ds(pl.multiple_of(g * 2 * D, 2 * D),
                                    2 * D)])


def _sc_readout(xf, bs):
    mesh = plsc.VectorSubcoreMesh(core_axis_name="c", subcore_axis_name="s")
    fn = functools.partial(
        pl.kernel, mesh=mesh,
        out_type=jax.ShapeDtypeStruct((G * 2 * D,), jnp.float32),
        scratch_types=[
            pltpu.VMEM((80,), jnp.int32),
            pltpu.VMEM((GSZ * D,), jnp.float32),
            pltpu.VMEM((2 * D,), jnp.float32),
            pltpu.SemaphoreType.DMA,
        ],
    )(_sc_readout_body)
    return fn(xf, bs)


def _pre_body(use_bn, x_ref, cs_ref, cq_ref, g_ref, bb_ref, p1_ref, p2_ref,
              cvec_ref, xhat_ref, a_ref, bt_ref):
    xv = x_ref[...]
    if use_bn:
        m = cs_ref[...] * (1.0 / N)
        v = cq_ref[...] * (1.0 / N) - m * m
        xv = (xv - m) / jnp.sqrt(v + 1e-5) * g_ref[...] + bb_ref[...]
        xv = jnp.maximum(xv, 0.0)
    xhat_ref[...] = xv
    a_ref[...] = jnp.dot(xv, p1_ref[...], preferred_element_type=jnp.float32)
    bt_ref[...] = (jnp.dot(xv, p2_ref[...], preferred_element_type=jnp.float32)
                   + cvec_ref[...])


def _tc_pre(x, cs, cq, gamma, beta, p1, p2, cvec, use_bn):
    row = lambda i: (i, 0)
    full = lambda i: (0, 0)
    return pl.pallas_call(
        functools.partial(_pre_body, use_bn),
        grid=(N // BLK,),
        in_specs=[
            pl.BlockSpec((BLK, D), row),
            pl.BlockSpec((1, D), full),
            pl.BlockSpec((1, D), full),
            pl.BlockSpec((1, D), full),
            pl.BlockSpec((1, D), full),
            pl.BlockSpec((D, D), full),
            pl.BlockSpec((D, D), full),
            pl.BlockSpec((1, D), full),
        ],
        out_specs=[
            pl.BlockSpec((BLK, D), row),
            pl.BlockSpec((BLK, D), row),
            pl.BlockSpec((BLK, D), row),
        ],
        out_shape=[jax.ShapeDtypeStruct((N, D), jnp.float32)] * 3,
    )(x, cs, cq, gamma, beta, p1, p2, cvec)


def _post_body(x_ref, a_ref, s1_ref, s2_ref, tn_ref, tx_ref, cnt_ref,
               q0_ref, q1_ref, q2_ref, q3_ref, pb_ref, lw_ref, lb_ref,
               y_ref, cs_ref, cq_ref):
    i = pl.program_id(0)
    a = a_ref[...]
    s1 = s1_ref[...]
    s2 = s2_ref[...]
    cnt = cnt_ref[...]
    cntc = jnp.maximum(cnt, 1.0)
    inv = 1.0 / cntc
    mean = (cnt * a + s1) * inv
    s2m = (cnt * a * a + 2.0 * a * s1 + s2) * inv
    var = s2m - mean * mean
    std = jnp.sqrt(jnp.maximum(var, 1e-5))
    has = cnt > 0.0
    mn = jnp.where(has, a + tn_ref[...], 0.0)
    mx = jnp.where(has, a + tx_ref[...], 0.0)
    A = jnp.concatenate([mean, mn, mx, std], axis=-1)
    lg = jnp.log(cntc + 1.0)
    amp = lg * (1.0 / AVG_LOG)
    att = AVG_LOG / lg
    dot = lambda u, w: jnp.dot(u, w[...], preferred_element_type=jnp.float32)
    z = (dot(x_ref[...], q0_ref) + dot(A, q1_ref) + dot(A * amp, q2_ref)
         + dot(A * att, q3_ref) + pb_ref[...])
    y = dot(z, lw_ref) + lb_ref[...]
    y_ref[...] = y

    @pl.when(i == 0)
    def _():
        cs_ref[...] = jnp.zeros_like(cs_ref)
        cq_ref[...] = jnp.zeros_like(cq_ref)

    cs_ref[...] += jnp.sum(y, axis=0, keepdims=True)
    cq_ref[...] += jnp.sum(y * y, axis=0, keepdims=True)


def _tc_post(x, a, s1, s2, tn, tx, cnt, q0, q1, q2, q3, pb, lw, lb):
    row = lambda i: (i, 0)
    full = lambda i: (0, 0)
    return pl.pallas_call(
        _post_body,
        grid=(N // BLK,),
        in_specs=[
            pl.BlockSpec((BLK, D), row),
            pl.BlockSpec((BLK, D), row),
            pl.BlockSpec((BLK, D), row),
            pl.BlockSpec((BLK, D), row),
            pl.BlockSpec((BLK, D), row),
            pl.BlockSpec((BLK, D), row),
            pl.BlockSpec((BLK, 1), row),
            pl.BlockSpec((D, D), full),
            pl.BlockSpec((4 * D, D), full),
            pl.BlockSpec((4 * D, D), full),
            pl.BlockSpec((4 * D, D), full),
            pl.BlockSpec((1, D), full),
            pl.BlockSpec((D, D), full),
            pl.BlockSpec((1, D), full),
        ],
        out_specs=[
            pl.BlockSpec((BLK, D), row),
            pl.BlockSpec((1, D), full),
            pl.BlockSpec((1, D), full),
        ],
        out_shape=[
            jax.ShapeDtypeStruct((N, D), jnp.float32),
            jax.ShapeDtypeStruct((1, D), jnp.float32),
            jax.ShapeDtypeStruct((1, D), jnp.float32),
        ],
    )(x, a, s1, s2, tn, tx, cnt, q0, q1, q2, q3, pb, lw, lb)


def _norm_body(x_ref, cs_ref, cq_ref, g_ref, bb_ref, o_ref):
    m = cs_ref[...] * (1.0 / N)
    v = cq_ref[...] * (1.0 / N) - m * m
    o_ref[...] = jnp.maximum(
        (x_ref[...] - m) / jnp.sqrt(v + 1e-5) * g_ref[...] + bb_ref[...], 0.0)


def _tc_norm(x, cs, cq, gamma, beta):
    row = lambda i: (i, 0)
    full = lambda i: (0, 0)
    return pl.pallas_call(
        _norm_body,
        grid=(N // BLK,),
        in_specs=[
            pl.BlockSpec((BLK, D), row),
            pl.BlockSpec((1, D), full),
            pl.BlockSpec((1, D), full),
            pl.BlockSpec((1, D), full),
            pl.BlockSpec((1, D), full),
        ],
        out_specs=pl.BlockSpec((BLK, D), row),
        out_shape=jax.ShapeDtypeStruct((N, D), jnp.float32),
    )(x, cs, cq, gamma, beta)


def _mlp_body(xa_ref, xs_ref, cg_ref, w1_ref, b1_ref, w2_ref, b2_ref,
              w3_ref, b3_ref, o_ref):
    cg = cg_ref[...]
    xa = jnp.where(cg > 0.0, xa_ref[...], 0.0)
    xb = xs_ref[...] / jnp.maximum(cg, 1.0)
    h = jnp.concatenate([xa, xb], axis=1)
    dot = lambda u, w: jnp.dot(u, w[...], preferred_element_type=jnp.float32)
    h = jnp.maximum(dot(h, w1_ref) + b1_ref[...], 0.0)
    h = jnp.maximum(dot(h, w2_ref) + b2_ref[...], 0.0)
    o_ref[...] = dot(h, w3_ref) + b3_ref[...]


def _tc_mlp(xa, xs, cg, w1, b1, w2, b2, w3p, b3p):
    return pl.pallas_call(
        _mlp_body,
        out_shape=jax.ShapeDtypeStruct((G, D), jnp.float32),
    )(xa, xs, cg, w1, b1, w2, b2, w3p, b3p)


def kernel(x, edge_index, edge_attr, intarna_energy, batch, covalent_edges,
           dropout_conv_1_2, dropout_conv_rest,
           c1_eW, c1_eb, c1_preW, c1_preb, c1_postW, c1_postb, c1_linW,
           c1_linb, c1_gamma, c1_beta,
           c2_eW, c2_eb, c2_preW, c2_preb, c2_postW, c2_postb, c2_linW,
           c2_linb, c2_gamma, c2_beta,
           c3_eW, c3_eb, c3_preW, c3_preb, c3_postW, c3_postb, c3_linW,
           c3_linb, c3_gamma, c3_beta,
           c4_eW, c4_eb, c4_preW, c4_preb, c4_postW, c4_postb, c4_linW,
           c4_linb, c4_gamma, c4_beta,
           lin1_W, lin1_b, lin2_W, lin2_b, lin3_W, lin3_b):
    src = edge_index[0]
    dst = edge_index[1]

    key = (dst.astype(jnp.uint32) << 18) | jnp.arange(E, dtype=jnp.uint32)
    skey = jax.lax.sort(key)
    perm = (skey & jnp.uint32((1 << 18) - 1)).astype(jnp.int32)
    sdst = (skey >> 18).astype(jnp.int32)
    row_start = jnp.searchsorted(sdst, jnp.arange(N + 1, dtype=jnp.int32))
    cnt = (row_start[1:] - row_start[:-1]).astype(jnp.float32)[:, None]

    srcp = jnp.concatenate([src[perm].astype(jnp.int32),
                            jnp.zeros((EP - E,), jnp.int32)])
    sdstp = jnp.concatenate([sdst, jnp.full((EP - E,), NP, jnp.int32)])
    eap = jnp.concatenate([edge_attr[perm],
                           jnp.zeros((EP - E, 4),
                                     jnp.float32)]).reshape(4 * EP)
    rsp = jnp.concatenate([
        row_start.astype(jnp.int32),
        jnp.full((RP - (N + 1),), E, jnp.int32)])
    zer = jnp.zeros((GSZ * D,), jnp.float32)

    layers = [
        (c1_eW, c1_eb, c1_preW, c1_preb, c1_postW, c1_postb, c1_linW,
         c1_linb, c1_gamma, c1_beta),
        (c2_eW, c2_eb, c2_preW, c2_preb, c2_postW, c2_postb, c2_linW,
         c2_linb, c2_gamma, c2_beta),
        (c3_eW, c3_eb, c3_preW, c3_preb, c3_postW, c3_postb, c3_linW,
         c3_linb, c3_gamma, c3_beta),
        (c4_eW, c4_eb, c4_preW, c4_preb, c4_postW, c4_postb, c4_linW,
         c4_linb, c4_gamma, c4_beta),
    ]

    r2 = lambda v: v.reshape(1, D)
    xcur = x
    cs = jnp.zeros((1, D), jnp.float32)
    cq = jnp.zeros((1, D), jnp.float32)
    for i, (eW, eb, preW, preb, postW, postb, linW, linb, gamma,
            beta) in enumerate(layers):
        p1 = preW[:D]
        p2 = preW[D:2 * D]
        p3 = preW[2 * D:]
        w4 = eW @ p3
        cvec = (eb @ p3 + preb).reshape(1, D)
        q0 = postW[:D]
        q1 = postW[D:5 * D]
        q2 = postW[5 * D:9 * D]
        q3 = postW[9 * D:]

        xhat, a, btab = _tc_pre(xcur, cs, cq, r2(gamma), r2(beta), p1, p2,
                                cvec, use_bn=(i > 0))

        s1, s2, tn, tx = _sc_stats(btab, srcp, sdstp, eap, rsp,
                                   w4.reshape(4 * D), zer)
        s1 = s1.reshape(NP, D)[:N]
        s2 = s2.reshape(NP, D)[:N]
        tn = tn.reshape(NP, D)[:N]
        tx = tx.reshape(NP, D)[:N]

        y, cs, cq = _tc_post(xhat, a, s1, s2, tn, tx, cnt, q0, q1, q2, q3,
                             postb.reshape(1, D), linW, linb.reshape(1, D))
        xcur = y

    xfin = _tc_norm(xcur, cs, cq, r2(layers[3][8]), r2(layers[3][9]))

    bstart = jnp.searchsorted(batch.astype(jnp.int32),
                              jnp.arange(G + 1, dtype=jnp.int32))
    cg = (bstart[1:] - bstart[:-1]).astype(jnp.float32)[:, None]
    bsp = jnp.concatenate([bstart.astype(jnp.int32),
                           jnp.full((80 - (G + 1),), N, jnp.int32)])
    xfp = jnp.concatenate([xfin, jnp.zeros((GSZ, D),
                                           jnp.float32)]).reshape(-1)
    xro = _sc_readout(xfp, bsp).reshape(G, 2, D)
    xa = xro[:, 0, :]
    xs = xro[:, 1, :]

    w3p = jnp.pad(lin3_W, ((0, 0), (0, D - 2)))
    b3p = jnp.pad(lin3_b, (0, D - 2)).reshape(1, D)
    out = _tc_mlp(xa, xs, cg, lin1_W, lin1_b.reshape(1, D), lin2_W,
                  lin2_b.reshape(1, 64), w3p, b3p)
    return out[:, :2]
